# Initial kernel scaffold; baseline (speedup 1.0000x reference)
import jax
import jax.numpy as jnp
from jax.experimental import pallas as pl


def kernel(x):
    raise NotImplementedError("write your pallas kernel here")


# speedup vs baseline: 13.8271x; 13.8271x over previous
import functools

import jax
import jax.numpy as jnp
from jax import lax
from jax.experimental import pallas as pl
from jax.experimental.pallas import tpu as pltpu
from jax.experimental.pallas import tpu_sc as plsc

_B = 128
_DIM = 32768
_M = 256
_NC = 2
_NS = 16
_NW = _NC * _NS
_ROWS_PER_W = _B // _NW
_L = 16
_NVEC = _DIM // _L

_SIGN = 0x80000000


def _u32_to_f32(u):
    sign = jnp.uint32(_SIGN)
    bits = jnp.where(u >= sign, u ^ sign, ~u)
    return lax.bitcast_convert_type(bits, jnp.float32)


@functools.partial(
    pl.kernel,
    out_type=jax.ShapeDtypeStruct((_B, _DIM), jnp.float32),
    mesh=plsc.VectorSubcoreMesh(
        core_axis_name="c", subcore_axis_name="s",
        num_cores=_NC, num_subcores=_NS),
    scratch_types=[pltpu.VMEM((_DIM,), jnp.float32)],
)
def _sc_topk_mask(x_hbm, out_hbm, row_v):
    wid = lax.axis_index("s") * _NC + lax.axis_index("c")
    ones = jnp.ones((_L,), jnp.int32)
    zeros = jnp.zeros((_L,), jnp.int32)
    zf = jnp.zeros((_L,), jnp.float32)

    for j in range(_ROWS_PER_W):
        row = wid * _ROWS_PER_W + j
        pltpu.sync_copy(x_hbm.at[row], row_v)

        def bit_step(i, prefix):
            bit = jnp.uint32(31) - i.astype(jnp.uint32)
            cand = prefix | (jnp.uint32(1) << bit)
            candf = jnp.full((_L,), _u32_to_f32(cand), dtype=jnp.float32)

            def cbody(k, acc):
                v = row_v[pl.ds(k * _L, _L)]
                return acc + jnp.where(v < candf, ones, zeros)

            acc = lax.fori_loop(0, _NVEC, cbody, zeros, unroll=8)
            cnt = acc[0]
            for t in range(1, _L):
                cnt = cnt + acc[t]
            return jnp.where(cnt >= _M, prefix, cand)

        tu = lax.fori_loop(0, 32, bit_step, jnp.uint32(0))
        tf = jnp.full((_L,), _u32_to_f32(tu), dtype=jnp.float32)

        def mbody(k, carry):
            v = row_v[pl.ds(k * _L, _L)]
            row_v[pl.ds(k * _L, _L)] = jnp.where(v <= tf, zf, v)
            return carry

        lax.fori_loop(0, _NVEC, mbody, 0, unroll=8)
        pltpu.sync_copy(row_v, out_hbm.at[row])


def kernel(x):
    return _sc_topk_mask(x)


# speedup vs baseline: 30.8734x; 2.2328x over previous
import functools

import jax
import jax.numpy as jnp
from jax import lax
from jax.experimental import pallas as pl
from jax.experimental.pallas import tpu as pltpu
from jax.experimental.pallas import tpu_sc as plsc

_B = 128
_DIM = 32768
_M = 256
_NC = 2
_NS = 16
_NW = _NC * _NS
_ROWS_PER_W = _B // _NW
_L = 16
_NVEC = _DIM // _L

_SAMPLE_VECS = 512
_INTERP_PASSES = 8
_TOTAL_PASSES = _INTERP_PASSES + 32
_CNT_UNROLL = 8
_MASK_UNROLL = 4

_SIGN = 0x80000000
_UHI_INIT = 0xFF800000
_UHI_INIT_I32 = _UHI_INIT - (1 << 32)

_C_CAND = 0
_C_ULO = 1
_C_UHI = 2
_C_LOCNT = 3
_C_HICNT = 4
_C_DONE = 5
_C_BOUND = 6


def _f32_to_u(f):
    b = lax.bitcast_convert_type(f, jnp.uint32)
    return jnp.where(b >= jnp.uint32(_SIGN), ~b, b | jnp.uint32(_SIGN))


def _u_to_f32(u):
    sign = jnp.uint32(_SIGN)
    bits = jnp.where(u >= sign, u ^ sign, ~u)
    return lax.bitcast_convert_type(bits, jnp.float32)


def _rcp(d):
    db = lax.bitcast_convert_type(d, jnp.int32)
    r = lax.bitcast_convert_type(jnp.int32(0x7EF311C3) - db, jnp.float32)
    r = r * (jnp.float32(2.0) - d * r)
    r = r * (jnp.float32(2.0) - d * r)
    return r


def _lane_sum_i(acc):
    s = acc[0]
    for t in range(1, _L):
        s = s + acc[t]
    return s


def _lane_sum_f(acc):
    s = acc[0]
    for t in range(1, _L):
        s = s + acc[t]
    return s


def _count_below(buf, cand_f):
    cvec = jnp.full((_L,), cand_f, dtype=jnp.float32)
    ione = jnp.ones((_L,), jnp.int32)
    izero = jnp.zeros((_L,), jnp.int32)

    def cbody(k, acc):
        base = k * (_L * _CNT_UNROLL)
        cs = [
            jnp.where(buf[pl.ds(base + t * _L, _L)] < cvec, ione, izero)
            for t in range(_CNT_UNROLL)
        ]
        while len(cs) > 1:
            cs = [cs[i] + cs[i + 1] for i in range(0, len(cs), 2)]
        return acc + cs[0]

    acc = lax.fori_loop(0, _NVEC // _CNT_UNROLL, cbody, izero, unroll=2)
    return _lane_sum_i(acc)


def _seed_candidate(buf):
    zf = jnp.zeros((_L,), jnp.float32)

    def sbody(k, c):
        s, a = c
        v = buf[pl.ds(k * _L, _L)]
        return (s + v, a + jnp.abs(v))

    sv, av = lax.fori_loop(0, _SAMPLE_VECS, sbody, (zf, zf), unroll=4)
    inv_n = jnp.float32(1.0 / (_SAMPLE_VECS * _L))
    mu = _lane_sum_f(sv) * inv_n
    mad = _lane_sum_f(av) * inv_n
    sd = mad * jnp.float32(1.2533141)
    return mu + sd * jnp.float32(-2.4177)


def _find_bound(buf, sm):
    c0u = _f32_to_u(_seed_candidate(buf))
    c0u = jnp.clip(c0u, jnp.uint32(1), jnp.uint32(_UHI_INIT - 1))
    sm[_C_CAND] = c0u.astype(jnp.int32)
    sm[_C_ULO] = jnp.int32(0)
    sm[_C_UHI] = jnp.int32(_UHI_INIT_I32)
    sm[_C_LOCNT] = jnp.int32(0)
    sm[_C_HICNT] = jnp.int32(_DIM)
    sm[_C_DONE] = jnp.int32(0)
    sm[_C_BOUND] = jnp.int32(_UHI_INIT_I32)

    def pbody(p, carry):
        @pl.when(sm[_C_DONE] == 0)
        def _():
            cand = sm[_C_CAND].astype(jnp.uint32)
            cnt = _count_below(buf, _u_to_f32(cand))
            ulo = sm[_C_ULO].astype(jnp.uint32)
            uhi = sm[_C_UHI].astype(jnp.uint32)
            locnt = sm[_C_LOCNT]
            hicnt = sm[_C_HICNT]

            hit = cnt == _M
            low = cnt < _M
            ulo2 = jnp.where(low, cand, ulo)
            locnt2 = jnp.where(low, cnt, locnt)
            uhi2 = jnp.where(low, uhi, cand)
            hicnt2 = jnp.where(low, hicnt, cnt)
            width1 = (uhi2 - ulo2) <= jnp.uint32(1)
            done = hit | width1
            bound = jnp.where(hit, cand, uhi2)

            du = (uhi2 - ulo2).astype(jnp.float32)
            num = (jnp.int32(_M) - locnt2).astype(jnp.float32)
            den = (hicnt2 - locnt2).astype(jnp.float32)
            step = (du * (num * _rcp(den))).astype(jnp.uint32)
            cand_i = ulo2 + step
            cand_b = ulo2 + ((uhi2 - ulo2) >> 1)
            nxt = jnp.where(p < _INTERP_PASSES, cand_i, cand_b)
            nxt = jnp.clip(nxt, ulo2 + jnp.uint32(1), uhi2 - jnp.uint32(1))

            sm[_C_CAND] = nxt.astype(jnp.int32)
            sm[_C_ULO] = ulo2.astype(jnp.int32)
            sm[_C_UHI] = uhi2.astype(jnp.int32)
            sm[_C_LOCNT] = locnt2
            sm[_C_HICNT] = hicnt2
            sm[_C_DONE] = jnp.where(done, jnp.int32(1), jnp.int32(0))
            sm[_C_BOUND] = bound.astype(jnp.int32)

        return carry

    lax.fori_loop(0, _TOTAL_PASSES, pbody, jnp.int32(0))
    return _u_to_f32(sm[_C_BOUND].astype(jnp.uint32))


def _mask_below(buf, bound_f):
    bvec = jnp.full((_L,), bound_f, dtype=jnp.float32)
    zf = jnp.zeros((_L,), jnp.float32)

    def mbody(k, carry):
        base = k * (_L * _MASK_UNROLL)
        for t in range(_MASK_UNROLL):
            v = buf[pl.ds(base + t * _L, _L)]
            buf[pl.ds(base + t * _L, _L)] = jnp.where(v < bvec, zf, v)
        return carry

    lax.fori_loop(0, _NVEC // _MASK_UNROLL, mbody, jnp.int32(0))


@functools.partial(
    pl.kernel,
    out_type=jax.ShapeDtypeStruct((_B, _DIM), jnp.float32),
    mesh=plsc.VectorSubcoreMesh(
        core_axis_name="c", subcore_axis_name="s",
        num_cores=_NC, num_subcores=_NS),
    scratch_types=[
        pltpu.VMEM((_DIM,), jnp.float32),
        pltpu.VMEM((_DIM,), jnp.float32),
        pltpu.VMEM((_DIM,), jnp.float32),
        pltpu.SMEM((8,), jnp.int32),
        pltpu.SemaphoreType.DMA,
        pltpu.SemaphoreType.DMA,
        pltpu.SemaphoreType.DMA,
    ],
)
def _sc_topk_mask(x_hbm, out_hbm, b0, b1, b2, sm, s0, s1, s2):
    wid = lax.axis_index("s") * _NC + lax.axis_index("c")
    rows = [wid * _ROWS_PER_W + j for j in range(_ROWS_PER_W)]
    bufs = [b0, b1, b2]
    sems = [s0, s1, s2]

    in_cp = [
        pltpu.async_copy(x_hbm.at[rows[j]], bufs[j], sems[j])
        for j in range(3)
    ]
    out_cp = [None] * _ROWS_PER_W
    for j in range(_ROWS_PER_W):
        s = j % 3
        in_cp[s].wait()
        bound_f = _find_bound(bufs[s], sm)
        _mask_below(bufs[s], bound_f)
        out_cp[j] = pltpu.async_copy(bufs[s], out_hbm.at[rows[j]], sems[s])
        if j == 1:
            out_cp[0].wait()
            in_cp[0] = pltpu.async_copy(
                x_hbm.at[rows[3]], bufs[0], sems[0])
    for j in range(1, _ROWS_PER_W):
        out_cp[j].wait()


def kernel(x):
    return _sc_topk_mask(x)


# speedup vs baseline: 38.6308x; 1.2513x over previous
import functools

import jax
import jax.numpy as jnp
from jax import lax
from jax.experimental import pallas as pl
from jax.experimental.pallas import tpu as pltpu
from jax.experimental.pallas import tpu_sc as plsc

_B = 128
_DIM = 32768
_M = 256
_NC = 2
_NS = 16
_NW = _NC * _NS
_ROWS_PER_W = _B // _NW
_L = 16
_NVEC = _DIM // _L

_SAMPLE_VECS = 512
_INTERP_PASSES = 8
_TOTAL_PASSES = _INTERP_PASSES + 32
_CNT_UNROLL = 16
_MASK_UNROLL = 8

_SIGN = 0x80000000
_UHI_INIT = 0xFF800000
_UHI_INIT_I32 = _UHI_INIT - (1 << 32)

_C_CAND = 0
_C_ULO = 1
_C_UHI = 2
_C_LOCNT = 3
_C_HICNT = 4
_C_DONE = 5
_C_BOUND = 6


def _f32_to_u(f):
    b = lax.bitcast_convert_type(f, jnp.uint32)
    return jnp.where(b >= jnp.uint32(_SIGN), ~b, b | jnp.uint32(_SIGN))


def _u_to_f32(u):
    sign = jnp.uint32(_SIGN)
    bits = jnp.where(u >= sign, u ^ sign, ~u)
    return lax.bitcast_convert_type(bits, jnp.float32)


def _rcp(d):
    db = lax.bitcast_convert_type(d, jnp.int32)
    r = lax.bitcast_convert_type(jnp.int32(0x7EF311C3) - db, jnp.float32)
    r = r * (jnp.float32(2.0) - d * r)
    r = r * (jnp.float32(2.0) - d * r)
    return r


def _lane_sum_i(acc):
    s = acc[0]
    for t in range(1, _L):
        s = s + acc[t]
    return s


def _lane_sum_f(acc):
    s = acc[0]
    for t in range(1, _L):
        s = s + acc[t]
    return s


def _count_below(buf, cand_f):
    cvec = jnp.full((_L,), cand_f, dtype=jnp.float32)
    ione = jnp.ones((_L,), jnp.int32)
    izero = jnp.zeros((_L,), jnp.int32)

    def cbody(k, acc):
        base = k * (_L * _CNT_UNROLL)
        cs = [
            jnp.where(buf[pl.ds(base + t * _L, _L)] < cvec, ione, izero)
            for t in range(_CNT_UNROLL)
        ]
        while len(cs) > 1:
            cs = [cs[i] + cs[i + 1] for i in range(0, len(cs), 2)]
        return acc + cs[0]

    acc = lax.fori_loop(0, _NVEC // _CNT_UNROLL, cbody, izero, unroll=2)
    return _lane_sum_i(acc)


def _seed_candidate(buf):
    zf = jnp.zeros((_L,), jnp.float32)

    def sbody(k, c):
        s, a = c
        v = buf[pl.ds(k * _L, _L)]
        return (s + v, a + jnp.abs(v))

    sv, av = lax.fori_loop(0, _SAMPLE_VECS, sbody, (zf, zf), unroll=4)
    inv_n = jnp.float32(1.0 / (_SAMPLE_VECS * _L))
    mu = _lane_sum_f(sv) * inv_n
    mad = _lane_sum_f(av) * inv_n
    sd = mad * jnp.float32(1.2533141)
    rho_inv = sd * jnp.float32(0.001471)
    return mu + sd * jnp.float32(-2.4177), rho_inv


def _find_bound(buf, sm):
    c0f, rho_inv = _seed_candidate(buf)
    c0u = jnp.clip(_f32_to_u(c0f),
                   jnp.uint32(1), jnp.uint32(_UHI_INIT - 1))
    sm[_C_CAND] = c0u.astype(jnp.int32)
    sm[_C_ULO] = jnp.int32(0)
    sm[_C_UHI] = jnp.int32(_UHI_INIT_I32)
    sm[_C_LOCNT] = jnp.int32(0)
    sm[_C_HICNT] = jnp.int32(_DIM)
    sm[_C_DONE] = jnp.int32(0)
    sm[_C_BOUND] = jnp.int32(_UHI_INIT_I32)

    def pbody(p, carry):
        @pl.when(sm[_C_DONE] == 0)
        def _():
            cand = sm[_C_CAND].astype(jnp.uint32)
            cand_f = _u_to_f32(cand)
            cnt = _count_below(buf, cand_f)
            ulo = sm[_C_ULO].astype(jnp.uint32)
            uhi = sm[_C_UHI].astype(jnp.uint32)
            locnt = sm[_C_LOCNT]
            hicnt = sm[_C_HICNT]

            hit = cnt == _M
            low = cnt < _M
            ulo2 = jnp.where(low, cand, ulo)
            locnt2 = jnp.where(low, cnt, locnt)
            uhi2 = jnp.where(low, uhi, cand)
            hicnt2 = jnp.where(low, hicnt, cnt)
            width1 = (uhi2 - ulo2) <= jnp.uint32(1)
            done = hit | width1
            bound = jnp.where(hit, cand, uhi2)

            du = (uhi2 - ulo2).astype(jnp.float32)
            num = (jnp.int32(_M) - locnt2).astype(jnp.float32)
            den = (hicnt2 - locnt2).astype(jnp.float32)
            step = (du * (num * _rcp(den))).astype(jnp.uint32)
            cand_i = ulo2 + step
            cand_b = ulo2 + ((uhi2 - ulo2) >> 1)
            model_f = cand_f + (jnp.int32(_M) - cnt).astype(jnp.float32) * rho_inv
            cand_m = _f32_to_u(model_f)
            open_bracket = (ulo2 == jnp.uint32(0)) | (
                uhi2 == jnp.uint32(_UHI_INIT))
            interp = p + 1 < _INTERP_PASSES
            nxt = jnp.where(interp & open_bracket, cand_m,
                            jnp.where(interp, cand_i, cand_b))
            nxt = jnp.clip(nxt, ulo2 + jnp.uint32(1), uhi2 - jnp.uint32(1))

            sm[_C_CAND] = nxt.astype(jnp.int32)
            sm[_C_ULO] = ulo2.astype(jnp.int32)
            sm[_C_UHI] = uhi2.astype(jnp.int32)
            sm[_C_LOCNT] = locnt2
            sm[_C_HICNT] = hicnt2
            sm[_C_DONE] = jnp.where(done, jnp.int32(1), jnp.int32(0))
            sm[_C_BOUND] = bound.astype(jnp.int32)

        return carry

    lax.fori_loop(0, _TOTAL_PASSES, pbody, jnp.int32(0))
    return _u_to_f32(sm[_C_BOUND].astype(jnp.uint32))


def _mask_below(buf, bound_f):
    bvec = jnp.full((_L,), bound_f, dtype=jnp.float32)
    zf = jnp.zeros((_L,), jnp.float32)

    def mbody(k, carry):
        base = k * (_L * _MASK_UNROLL)
        for t in range(_MASK_UNROLL):
            v = buf[pl.ds(base + t * _L, _L)]
            buf[pl.ds(base + t * _L, _L)] = jnp.where(v < bvec, zf, v)
        return carry

    lax.fori_loop(0, _NVEC // _MASK_UNROLL, mbody, jnp.int32(0), unroll=2)


@functools.partial(
    pl.kernel,
    out_type=jax.ShapeDtypeStruct((_B, _DIM), jnp.float32),
    mesh=plsc.VectorSubcoreMesh(
        core_axis_name="c", subcore_axis_name="s",
        num_cores=_NC, num_subcores=_NS),
    scratch_types=[
        pltpu.VMEM((_DIM,), jnp.float32),
        pltpu.VMEM((_DIM,), jnp.float32),
        pltpu.VMEM((_DIM,), jnp.float32),
        pltpu.SMEM((8,), jnp.int32),
        pltpu.SemaphoreType.DMA,
        pltpu.SemaphoreType.DMA,
        pltpu.SemaphoreType.DMA,
    ],
)
def _sc_topk_mask(x_hbm, out_hbm, b0, b1, b2, sm, s0, s1, s2):
    wid = lax.axis_index("s") * _NC + lax.axis_index("c")
    rows = [wid * _ROWS_PER_W + j for j in range(_ROWS_PER_W)]
    bufs = [b0, b1, b2]
    sems = [s0, s1, s2]

    in_cp = [
        pltpu.async_copy(x_hbm.at[rows[j]], bufs[j], sems[j])
        for j in range(3)
    ]
    out_cp = [None] * _ROWS_PER_W
    for j in range(_ROWS_PER_W):
        s = j % 3
        in_cp[s].wait()
        bound_f = _find_bound(bufs[s], sm)
        _mask_below(bufs[s], bound_f)
        out_cp[j] = pltpu.async_copy(bufs[s], out_hbm.at[rows[j]], sems[s])
        if j == 1:
            out_cp[0].wait()
            in_cp[0] = pltpu.async_copy(
                x_hbm.at[rows[3]], bufs[0], sems[0])
    for j in range(1, _ROWS_PER_W):
        out_cp[j].wait()


def kernel(x):
    return _sc_topk_mask(x)
